# baseline (speedup 1.0000x reference)
import functools

import jax
import jax.numpy as jnp
from jax import lax
from jax.experimental import pallas as pl
from jax.experimental.pallas import tpu as pltpu
from jax.experimental.pallas import tpu_sc as plsc

_B = 16384
_D = 32
_LN = int(_B * 0.1)
_CHUNK = 128


def _sc_gather(item_table, noise_table, user_table, idx_item, idx_noise, idx_user):
    info = plsc.get_sparse_core_info()
    nw = info.num_cores * info.num_subcores
    ci = idx_item.shape[0] // nw
    cn = idx_noise.shape[0] // nw
    cu = idx_user.shape[0] // nw
    mesh = plsc.VectorSubcoreMesh(core_axis_name="c", subcore_axis_name="s")

    @functools.partial(
        pl.kernel,
        mesh=mesh,
        compiler_params=pltpu.CompilerParams(use_tc_tiling_on_sc=False),
        out_type=[
            jax.ShapeDtypeStruct((idx_item.shape[0], _D), jnp.float32),
            jax.ShapeDtypeStruct((idx_noise.shape[0], _D), jnp.float32),
            jax.ShapeDtypeStruct((idx_user.shape[0], _D), jnp.float32),
        ],
        scratch_types=[
            pltpu.VMEM((ci,), jnp.int32),
            pltpu.VMEM((cn,), jnp.int32),
            pltpu.VMEM((cu,), jnp.int32),
            pltpu.VMEM((_CHUNK, _D), jnp.float32),
            pltpu.VMEM((_CHUNK, _D), jnp.float32),
            pltpu.SemaphoreType.DMA,
            pltpu.SemaphoreType.DMA,
        ],
    )
    def k(item_hbm, noise_hbm, user_hbm, ii_hbm, in_hbm, iu_hbm,
          oi_hbm, on_hbm, ou_hbm, ivi, ivn, ivu, rows_a, rows_b, sem_a, sem_b):
        w = lax.axis_index("s") * info.num_cores + lax.axis_index("c")

        def gather_list(idx_hbm, idx_v, nrows, table_hbm, out_hbm):
            base = w * nrows
            pltpu.sync_copy(idx_hbm.at[pl.ds(base, nrows)], idx_v)
            nchunks = nrows // _CHUNK
            bufs = (rows_a, rows_b)
            sems = (sem_a, sem_b)
            cps = []
            for c in range(nchunks):
                cp = pltpu.async_copy(
                    table_hbm.at[idx_v.at[pl.ds(c * _CHUNK, _CHUNK)]],
                    bufs[c % 2], sems[c % 2])
                cps.append(cp)
                if c >= 1:
                    cps[c - 1].wait()
                    pltpu.sync_copy(bufs[(c - 1) % 2],
                                    out_hbm.at[pl.ds(base + (c - 1) * _CHUNK, _CHUNK)])
            cps[nchunks - 1].wait()
            pltpu.sync_copy(bufs[(nchunks - 1) % 2],
                            out_hbm.at[pl.ds(base + (nchunks - 1) * _CHUNK, _CHUNK)])

        gather_list(ii_hbm, ivi, ci, item_hbm, oi_hbm)
        gather_list(in_hbm, ivn, cn, noise_hbm, on_hbm)
        gather_list(iu_hbm, ivu, cu, user_hbm, ou_hbm)

    return k(item_table, noise_table, user_table, idx_item, idx_noise, idx_user)


def _softplus(x):
    return jnp.maximum(x, 0.0) + jnp.log1p(jnp.exp(-jnp.abs(x)))


_ROWS_PER_STEP = 2048
_GRID = _B // _ROWS_PER_STEP


def _loss_kernel(i_ref, j_ref, jr_ref, lki_ref, ni_ref, njr_ref, lkn_ref,
                 u_ref, m_ref, out_ref):
    pid = pl.program_id(0)
    row0 = pid * _ROWS_PER_STEP
    it = lax.broadcasted_iota(jnp.int32, (_ROWS_PER_STEP, 1), 0) + row0

    u = u_ref[...]
    ib = i_ref[...]
    jb = j_ref[...]
    jr = jr_ref[...]
    lki = lki_ref[...]
    nib = ni_ref[...]
    njr = njr_ref[...]
    lkn = lkn_ref[...]
    m = m_ref[...]

    noise_i = jnp.clip(ib, -1.0, 1.0) + nib
    add = jnp.where(it < _B - _LN, ib, noise_i)
    addj = jnp.where(it < _LN, jnp.clip(jr, -1.0, 1.0) + njr, jr)
    like = jnp.clip(lki, -1.0, 1.0) + lkn

    pa = jnp.sum(u * add, axis=1, keepdims=True)
    pn = jnp.sum(u * addj, axis=1, keepdims=True)
    s_add = jnp.sum(_softplus(pn - pa))

    l2row = jnp.sum(u * u + add * add + jb * jb, axis=1, keepdims=True)
    s_l2 = jnp.sum(l2row)

    dm = jnp.sum(u * (like - jb), axis=1, keepdims=True)
    spm = _softplus(-dm)
    s_fm = jnp.sum(spm * m)
    s_ff = jnp.sum(spm * (1.0 - m))

    lk2 = jnp.sum(like * like, axis=1, keepdims=True)
    s_lm = jnp.sum(lk2 * m)
    s_lf = jnp.sum(lk2 * (1.0 - m))

    zero = jnp.float32(0.0)
    vals = jnp.stack([jnp.broadcast_to(s, (128,)) for s in
                      (s_add, s_l2, s_fm, s_ff, s_lm, s_lf, zero, zero)])

    @pl.when(pid == 0)
    def _():
        out_ref[...] = jnp.zeros((8, 128), jnp.float32)

    out_ref[...] += vals


def kernel(u_batch, i_batch, j_batch, user_table, item_table, noise_table,
           users_features):
    u_batch = u_batch.astype(jnp.int32)
    i_batch = i_batch.astype(jnp.int32)
    j_batch = j_batch.astype(jnp.int32)

    gender = users_features[u_batch]
    male = gender.astype(bool)
    male_len = jnp.sum(gender)
    female_len = _B - male_len
    order_m = jnp.argsort(jnp.logical_not(male))
    order_f = jnp.argsort(male)
    rank_m = jnp.cumsum(gender) - 1
    rank_f = jnp.cumsum(1 - gender) - 1
    mm = jnp.maximum(male_len - 1, 1)
    fm = jnp.maximum(female_len - 1, 1)
    p_idx = i_batch[order_f[jnp.mod(rank_m, fm)]]
    q_idx = i_batch[order_m[jnp.mod(rank_f, mm)]]
    like_idx = jnp.where(male, p_idx, q_idx)

    j_roll = jnp.roll(j_batch, _LN)
    idx_item = jnp.concatenate([i_batch, j_batch, j_roll, like_idx])
    idx_noise = jnp.concatenate([i_batch, j_roll, like_idx])

    item_rows, noise_rows, u_rows = _sc_gather(
        item_table, noise_table, user_table, idx_item, idx_noise, u_batch)

    i_r = item_rows[:_B]
    j_r = item_rows[_B:2 * _B]
    jr_r = item_rows[2 * _B:3 * _B]
    lki_r = item_rows[3 * _B:]
    ni_r = noise_rows[:_B]
    njr_r = noise_rows[_B:2 * _B]
    lkn_r = noise_rows[2 * _B:]
    m_f = gender.astype(jnp.float32).reshape(_B, 1)

    bs = lambda shp: pl.BlockSpec(shp, lambda i: (i, 0))
    sums = pl.pallas_call(
        _loss_kernel,
        grid=(_GRID,),
        in_specs=[bs((_ROWS_PER_STEP, _D))] * 8 + [bs((_ROWS_PER_STEP, 1))],
        out_specs=pl.BlockSpec((8, 128), lambda i: (0, 0)),
        out_shape=jax.ShapeDtypeStruct((8, 128), jnp.float32),
    )(i_r, j_r, jr_r, lki_r, ni_r, njr_r, lkn_r, u_rows, m_f)

    s_add, s_l2, s_fm, s_ff, s_lm, s_lf = (sums[k, 0] for k in range(6))
    loss_add = s_add / _B
    l2_reg = 0.01 * s_l2 / _B
    loss_fake = s_fm / male_len + s_ff / female_len
    l2_reg2 = 0.01 * (s_lm / male_len + s_lf / female_len)
    return (loss_add + l2_reg, l2_reg, loss_fake + l2_reg2)


# speedup vs baseline: 1.0943x; 1.0943x over previous
import functools

import jax
import jax.numpy as jnp
from jax import lax
from jax.experimental import pallas as pl
from jax.experimental.pallas import tpu as pltpu
from jax.experimental.pallas import tpu_sc as plsc

_B = 16384
_D = 32
_LN = int(_B * 0.1)
_CHUNK = 128


def _sc_gather(item_table, noise_table, user_table, idx_item, idx_noise, idx_user):
    info = plsc.get_sparse_core_info()
    nw = info.num_cores * info.num_subcores
    ci = idx_item.shape[0] // nw
    cn = idx_noise.shape[0] // nw
    cu = idx_user.shape[0] // nw
    mesh = plsc.VectorSubcoreMesh(core_axis_name="c", subcore_axis_name="s")

    @functools.partial(
        pl.kernel,
        mesh=mesh,
        compiler_params=pltpu.CompilerParams(use_tc_tiling_on_sc=False),
        out_type=[
            jax.ShapeDtypeStruct((idx_item.shape[0], _D), jnp.float32),
            jax.ShapeDtypeStruct((idx_noise.shape[0], _D), jnp.float32),
            jax.ShapeDtypeStruct((idx_user.shape[0], _D), jnp.float32),
        ],
        scratch_types=[
            pltpu.VMEM((ci,), jnp.int32),
            pltpu.VMEM((cn,), jnp.int32),
            pltpu.VMEM((cu,), jnp.int32),
            pltpu.VMEM((_CHUNK, _D), jnp.float32),
            pltpu.VMEM((_CHUNK, _D), jnp.float32),
            pltpu.SemaphoreType.DMA,
            pltpu.SemaphoreType.DMA,
        ],
    )
    def k(item_hbm, noise_hbm, user_hbm, ii_hbm, in_hbm, iu_hbm,
          oi_hbm, on_hbm, ou_hbm, ivi, ivn, ivu, rows_a, rows_b, sem_a, sem_b):
        w = lax.axis_index("s") * info.num_cores + lax.axis_index("c")

        def gather_list(idx_hbm, idx_v, nrows, table_hbm, out_hbm):
            base = w * nrows
            pltpu.sync_copy(idx_hbm.at[pl.ds(base, nrows)], idx_v)
            nchunks = nrows // _CHUNK
            bufs = (rows_a, rows_b)
            sems = (sem_a, sem_b)
            cps = []
            for c in range(nchunks):
                cp = pltpu.async_copy(
                    table_hbm.at[idx_v.at[pl.ds(c * _CHUNK, _CHUNK)]],
                    bufs[c % 2], sems[c % 2])
                cps.append(cp)
                if c >= 1:
                    cps[c - 1].wait()
                    pltpu.sync_copy(bufs[(c - 1) % 2],
                                    out_hbm.at[pl.ds(base + (c - 1) * _CHUNK, _CHUNK)])
            cps[nchunks - 1].wait()
            pltpu.sync_copy(bufs[(nchunks - 1) % 2],
                            out_hbm.at[pl.ds(base + (nchunks - 1) * _CHUNK, _CHUNK)])

        gather_list(ii_hbm, ivi, ci, item_hbm, oi_hbm)
        gather_list(in_hbm, ivn, cn, noise_hbm, on_hbm)
        gather_list(iu_hbm, ivu, cu, user_hbm, ou_hbm)

    return k(item_table, noise_table, user_table, idx_item, idx_noise, idx_user)


def _softplus(x):
    return jnp.maximum(x, 0.0) + jnp.log1p(jnp.exp(-jnp.abs(x)))


_ROWS_PER_STEP = 2048
_GRID = _B // _ROWS_PER_STEP


def _loss_kernel(i_ref, j_ref, jr_ref, lki_ref, ni_ref, njr_ref, lkn_ref,
                 u_ref, m_ref, out_ref):
    pr = _ROWS_PER_STEP * _D // 128
    pid = pl.program_id(0)
    prow = lax.broadcasted_iota(jnp.int32, (pr, 128), 0) + pid * pr
    lane = lax.broadcasted_iota(jnp.int32, (pr, 128), 1)
    it = prow * 4 + lane // _D

    u = u_ref[...]
    ib = i_ref[...]
    jb = j_ref[...]
    jr = jr_ref[...]
    lki = lki_ref[...]
    nib = ni_ref[...]
    njr = njr_ref[...]
    lkn = lkn_ref[...]
    m4 = m_ref[...]

    li = lax.broadcasted_iota(jnp.int32, (128, 128), 0)
    gi = lax.broadcasted_iota(jnp.int32, (128, 128), 1)
    sel = ((li // _D) == gi).astype(jnp.float32)
    colv = (lane < 128 // _D).astype(jnp.float32)

    noise_i = jnp.clip(ib, -1.0, 1.0) + nib
    add = jnp.where(it < _B - _LN, ib, noise_i)
    addj = jnp.where(it < _LN, jnp.clip(jr, -1.0, 1.0) + njr, jr)
    like = jnp.clip(lki, -1.0, 1.0) + lkn

    d4 = jnp.dot(u * (add - addj), sel, preferred_element_type=jnp.float32)
    s_add = jnp.sum(_softplus(-d4) * colv)

    s_l2 = jnp.sum(u * u + add * add + jb * jb)

    dm4 = jnp.dot(u * (like - jb), sel, preferred_element_type=jnp.float32)
    spm = _softplus(-dm4)
    s_fm = jnp.sum(spm * m4)
    s_ff = jnp.sum(spm * (colv - m4))

    lk4 = jnp.dot(like * like, sel, preferred_element_type=jnp.float32)
    s_lm = jnp.sum(lk4 * m4)
    s_lf = jnp.sum(lk4 * (colv - m4))

    zero = jnp.float32(0.0)
    vals = jnp.stack([jnp.broadcast_to(s, (128,)) for s in
                      (s_add, s_l2, s_fm, s_ff, s_lm, s_lf, zero, zero)])

    @pl.when(pid == 0)
    def _():
        out_ref[...] = jnp.zeros((8, 128), jnp.float32)

    out_ref[...] += vals


def kernel(u_batch, i_batch, j_batch, user_table, item_table, noise_table,
           users_features):
    u_batch = u_batch.astype(jnp.int32)
    i_batch = i_batch.astype(jnp.int32)
    j_batch = j_batch.astype(jnp.int32)

    gender = users_features[u_batch]
    male = gender.astype(bool)
    male_len = jnp.sum(gender)
    female_len = _B - male_len
    order_m = jnp.argsort(jnp.logical_not(male))
    order_f = jnp.roll(order_m, female_len)
    rank_m = jnp.cumsum(gender) - 1
    rank_f = jnp.cumsum(1 - gender) - 1
    mm = jnp.maximum(male_len - 1, 1)
    fm = jnp.maximum(female_len - 1, 1)
    p_idx = i_batch[order_f[jnp.mod(rank_m, fm)]]
    q_idx = i_batch[order_m[jnp.mod(rank_f, mm)]]
    like_idx = jnp.where(male, p_idx, q_idx)

    j_roll = jnp.roll(j_batch, _LN)
    idx_item = jnp.concatenate([i_batch, j_batch, j_roll, like_idx])
    idx_noise = jnp.concatenate([i_batch, j_roll, like_idx])

    item_rows, noise_rows, u_rows = _sc_gather(
        item_table, noise_table, user_table, idx_item, idx_noise, u_batch)

    item_pack = item_rows.reshape(-1, 128)
    noise_pack = noise_rows.reshape(-1, 128)
    u_pack = u_rows.reshape(-1, 128)
    m_pack = jnp.pad(gender.astype(jnp.float32).reshape(-1, 128 // _D),
                     ((0, 0), (0, 128 - 128 // _D)))

    pr = _ROWS_PER_STEP * _D // 128
    seg = lambda k: pl.BlockSpec((pr, 128), lambda i, k=k: (k * _GRID + i, 0))
    sums = pl.pallas_call(
        _loss_kernel,
        grid=(_GRID,),
        in_specs=[seg(0), seg(1), seg(2), seg(3),
                  seg(0), seg(1), seg(2),
                  seg(0),
                  seg(0)],
        out_specs=pl.BlockSpec((8, 128), lambda i: (0, 0)),
        out_shape=jax.ShapeDtypeStruct((8, 128), jnp.float32),
    )(item_pack, item_pack, item_pack, item_pack,
      noise_pack, noise_pack, noise_pack, u_pack, m_pack)

    s_add, s_l2, s_fm, s_ff, s_lm, s_lf = (sums[k, 0] for k in range(6))
    loss_add = s_add / _B
    l2_reg = 0.01 * s_l2 / _B
    loss_fake = s_fm / male_len + s_ff / female_len
    l2_reg2 = 0.01 * (s_lm / male_len + s_lf / female_len)
    return (loss_add + l2_reg, l2_reg, loss_fake + l2_reg2)


# speedup vs baseline: 1.1016x; 1.0066x over previous
import functools

import jax
import jax.numpy as jnp
from jax import lax
from jax.experimental import pallas as pl
from jax.experimental.pallas import tpu as pltpu
from jax.experimental.pallas import tpu_sc as plsc

_B = 16384
_D = 32
_LN = int(_B * 0.1)
_CHUNK = 128


def _sc_gather(item_table, noise_table, user_table, idx_item, idx_noise, idx_user):
    info = plsc.get_sparse_core_info()
    nw = info.num_cores * info.num_subcores
    ci = idx_item.shape[0] // nw
    cn = idx_noise.shape[0] // nw
    cu = idx_user.shape[0] // nw
    mesh = plsc.VectorSubcoreMesh(core_axis_name="c", subcore_axis_name="s")

    @functools.partial(
        pl.kernel,
        mesh=mesh,
        compiler_params=pltpu.CompilerParams(use_tc_tiling_on_sc=False),
        out_type=[
            jax.ShapeDtypeStruct((idx_item.shape[0], _D), jnp.float32),
            jax.ShapeDtypeStruct((idx_noise.shape[0], _D), jnp.float32),
            jax.ShapeDtypeStruct((idx_user.shape[0], _D), jnp.float32),
        ],
        scratch_types=[
            pltpu.VMEM((ci,), jnp.int32),
            pltpu.VMEM((cn,), jnp.int32),
            pltpu.VMEM((cu,), jnp.int32),
            pltpu.VMEM((_CHUNK, _D), jnp.float32),
            pltpu.VMEM((_CHUNK, _D), jnp.float32),
            pltpu.SemaphoreType.DMA,
            pltpu.SemaphoreType.DMA,
        ],
    )
    def k(item_hbm, noise_hbm, user_hbm, ii_hbm, in_hbm, iu_hbm,
          oi_hbm, on_hbm, ou_hbm, ivi, ivn, ivu, rows_a, rows_b, sem_a, sem_b):
        w = lax.axis_index("s") * info.num_cores + lax.axis_index("c")

        def gather_list(idx_hbm, idx_v, nrows, table_hbm, out_hbm):
            base = w * nrows
            pltpu.sync_copy(idx_hbm.at[pl.ds(base, nrows)], idx_v)
            nchunks = nrows // _CHUNK
            bufs = (rows_a, rows_b)
            sems = (sem_a, sem_b)
            cps = []
            for c in range(nchunks):
                cp = pltpu.async_copy(
                    table_hbm.at[idx_v.at[pl.ds(c * _CHUNK, _CHUNK)]],
                    bufs[c % 2], sems[c % 2])
                cps.append(cp)
                if c >= 1:
                    cps[c - 1].wait()
                    pltpu.sync_copy(bufs[(c - 1) % 2],
                                    out_hbm.at[pl.ds(base + (c - 1) * _CHUNK, _CHUNK)])
            cps[nchunks - 1].wait()
            pltpu.sync_copy(bufs[(nchunks - 1) % 2],
                            out_hbm.at[pl.ds(base + (nchunks - 1) * _CHUNK, _CHUNK)])

        gather_list(ii_hbm, ivi, ci, item_hbm, oi_hbm)
        gather_list(in_hbm, ivn, cn, noise_hbm, on_hbm)
        gather_list(iu_hbm, ivu, cu, user_hbm, ou_hbm)

    return k(item_table, noise_table, user_table, idx_item, idx_noise, idx_user)


def _softplus(x):
    return jnp.maximum(x, 0.0) + jnp.log1p(jnp.exp(-jnp.abs(x)))


_ROWS_PER_STEP = 2048
_GRID = _B // _ROWS_PER_STEP


def _loss_kernel(i_ref, j_ref, jr_ref, lki_ref, ni_ref, njr_ref, lkn_ref,
                 u_ref, m_ref, out_ref):
    pr = _ROWS_PER_STEP * _D // 128
    pid = pl.program_id(0)
    prow = lax.broadcasted_iota(jnp.int32, (pr, 128), 0) + pid * pr
    lane = lax.broadcasted_iota(jnp.int32, (pr, 128), 1)
    it = prow * 4 + lane // _D

    u = u_ref[...]
    ib = i_ref[...]
    jb = j_ref[...]
    jr = jr_ref[...]
    lki = lki_ref[...]
    nib = ni_ref[...]
    njr = njr_ref[...]
    lkn = lkn_ref[...]
    m4 = m_ref[...]

    li = lax.broadcasted_iota(jnp.int32, (128, 128), 0)
    gi = lax.broadcasted_iota(jnp.int32, (128, 128), 1)
    sel = ((li // _D) == gi).astype(jnp.float32)
    colv = (lane < 128 // _D).astype(jnp.float32)

    noise_i = jnp.clip(ib, -1.0, 1.0) + nib
    add = jnp.where(it < _B - _LN, ib, noise_i)
    addj = jnp.where(it < _LN, jnp.clip(jr, -1.0, 1.0) + njr, jr)
    like = jnp.clip(lki, -1.0, 1.0) + lkn

    d4 = jnp.dot(u * (add - addj), sel, preferred_element_type=jnp.float32)
    s_add = jnp.sum(_softplus(-d4) * colv)

    s_l2 = jnp.sum(u * u + add * add + jb * jb)

    dm4 = jnp.dot(u * (like - jb), sel, preferred_element_type=jnp.float32)
    spm = _softplus(-dm4)
    s_fm = jnp.sum(spm * m4)
    s_ff = jnp.sum(spm * (colv - m4))

    lk4 = jnp.dot(like * like, sel, preferred_element_type=jnp.float32)
    s_lm = jnp.sum(lk4 * m4)
    s_lf = jnp.sum(lk4 * (colv - m4))

    zero = jnp.float32(0.0)
    vals = jnp.stack([jnp.broadcast_to(s, (128,)) for s in
                      (s_add, s_l2, s_fm, s_ff, s_lm, s_lf, zero, zero)])

    @pl.when(pid == 0)
    def _():
        out_ref[...] = jnp.zeros((8, 128), jnp.float32)

    out_ref[...] += vals


def kernel(u_batch, i_batch, j_batch, user_table, item_table, noise_table,
           users_features):
    u_batch = u_batch.astype(jnp.int32)
    i_batch = i_batch.astype(jnp.int32)
    j_batch = j_batch.astype(jnp.int32)

    gender = users_features[u_batch]
    male = gender.astype(bool)
    male_len = jnp.sum(gender)
    female_len = _B - male_len
    order_m = jnp.argsort(jnp.logical_not(male))
    cm = jnp.cumsum(gender)
    rank_m = cm - 1
    rank_f = jnp.arange(_B, dtype=cm.dtype) - cm
    mm = jnp.maximum(male_len - 1, 1)
    fm = jnp.maximum(female_len - 1, 1)
    like_pos = order_m[jnp.where(
        male,
        jnp.mod(jnp.mod(rank_m, fm) + male_len, _B),
        jnp.mod(rank_f, mm))]
    like_idx = i_batch[like_pos]

    j_roll = jnp.roll(j_batch, _LN)
    idx_item = jnp.concatenate([i_batch, j_batch, j_roll, like_idx])
    idx_noise = jnp.concatenate([i_batch, j_roll, like_idx])

    item_rows, noise_rows, u_rows = _sc_gather(
        item_table, noise_table, user_table, idx_item, idx_noise, u_batch)

    item_pack = item_rows.reshape(-1, 128)
    noise_pack = noise_rows.reshape(-1, 128)
    u_pack = u_rows.reshape(-1, 128)
    m_pack = jnp.pad(gender.astype(jnp.float32).reshape(-1, 128 // _D),
                     ((0, 0), (0, 128 - 128 // _D)))

    pr = _ROWS_PER_STEP * _D // 128
    seg = lambda k: pl.BlockSpec((pr, 128), lambda i, k=k: (k * _GRID + i, 0))
    sums = pl.pallas_call(
        _loss_kernel,
        grid=(_GRID,),
        in_specs=[seg(0), seg(1), seg(2), seg(3),
                  seg(0), seg(1), seg(2),
                  seg(0),
                  seg(0)],
        out_specs=pl.BlockSpec((8, 128), lambda i: (0, 0)),
        out_shape=jax.ShapeDtypeStruct((8, 128), jnp.float32),
    )(item_pack, item_pack, item_pack, item_pack,
      noise_pack, noise_pack, noise_pack, u_pack, m_pack)

    s_add, s_l2, s_fm, s_ff, s_lm, s_lf = (sums[k, 0] for k in range(6))
    loss_add = s_add / _B
    l2_reg = 0.01 * s_l2 / _B
    loss_fake = s_fm / male_len + s_ff / female_len
    l2_reg2 = 0.01 * (s_lm / male_len + s_lf / female_len)
    return (loss_add + l2_reg, l2_reg, loss_fake + l2_reg2)
